# baseline (speedup 1.0000x reference)
import functools

import jax
import jax.numpy as jnp
from jax import lax
from jax.experimental import pallas as pl
from jax.experimental.pallas import tpu as pltpu
from jax.experimental.pallas import tpu_sc as plsc

_N = 10000
_E = 160000
_EPS = 1e-5
_FB = 128
_CH = 128
_NT = 16
_EP = _E + _N
_CHUNK = _NT * _CH
_EPP = ((_EP + _CHUNK - 1) // _CHUNK) * _CHUNK
_PER_TILE = _EPP // _NT
_NCH = _PER_TILE // _CH
_EROWS = _EPP // _CH
_NACC = 10240
_RPT = _NACC // _NT
_BM = 2000


def _sc_mesh():
    return plsc.VectorSubcoreMesh(core_axis_name="c", subcore_axis_name="s")


def _deg_body(dst_hbm, deg_hbm, idx_v, ones_v, zbuf_v, acc_s, sem):
    cid = lax.axis_index("c")
    sid = lax.axis_index("s")
    half = _EPP // 2
    ppt = half // _NT

    def f1(i, c):
        ones_v[pl.ds(i * 16, 16)] = jnp.ones((16,), jnp.float32)
        return c

    lax.fori_loop(0, _CH // 16, f1, 0)

    def f0(i, c):
        zbuf_v[pl.ds(i * 16, 16)] = jnp.zeros((16,), jnp.float32)
        return c

    lax.fori_loop(0, _RPT // 16, f0, 0)
    pltpu.sync_copy(zbuf_v, acc_s.at[pl.ds(sid * _RPT, _RPT)])
    plsc.subcore_barrier()
    ebase = cid * half + sid * ppt

    def body(i, c):
        pltpu.sync_copy(dst_hbm.at[pl.ds(ebase + i * _CH, _CH)], idx_v)
        pltpu.sync_copy(ones_v, acc_s.at[idx_v], add=True)
        return c

    lax.fori_loop(0, ppt // _CH, body, 0)
    plsc.subcore_barrier()
    for c in range(2):
        @pl.when(cid == c)
        def _(c=c):
            pltpu.sync_copy(acc_s.at[pl.ds(sid * _RPT, _RPT)],
                            deg_hbm.at[c].at[pl.ds(sid * _RPT, _RPT)])


_deg_call = pl.kernel(
    _deg_body,
    out_type=jax.ShapeDtypeStruct((2, _NACC), jnp.float32),
    mesh=_sc_mesh(),
    scratch_types=[
        pltpu.VMEM((_CH,), jnp.int32),
        pltpu.VMEM((_CH,), jnp.float32),
        pltpu.VMEM((_RPT,), jnp.float32),
        pltpu.VMEM_SHARED((_NACC,), jnp.float32),
        pltpu.SemaphoreType.DMA,
    ],
)


def _make_prop(nb):
    bpc = nb // 2

    def body(src_hbm, dst_hbm, h_hbm, out_hbm, sb0, db0, sb1, db1, rows_v,
             acc_s, gsem0, gsem1, ssem0, ssem1):
        cid = lax.axis_index("c")
        sid = lax.axis_index("s")
        ebase = sid * _PER_TILE

        for fb in range(nb):
            owner = fb // bpc

            @pl.when(cid == owner)
            def _(fb=fb):
                def zf(i, c):
                    rows_v[0, i // (_FB // 16),
                           pl.ds((i % (_FB // 16)) * 16, 16)] = (
                        jnp.zeros((16,), jnp.float32))
                    return c

                lax.fori_loop(0, _CH * (_FB // 16), zf, 0)
                for z in range(_RPT // _CH):
                    pltpu.sync_copy(
                        rows_v.at[0],
                        acc_s.at[pl.ds(sid * _RPT + z * _CH, _CH)])
                plsc.subcore_barrier()
                hblk = h_hbm.at[fb]
                pltpu.sync_copy(src_hbm.at[pl.ds(ebase, _CH)], sb0)
                pltpu.sync_copy(dst_hbm.at[pl.ds(ebase, _CH)], db0)
                pltpu.async_copy(hblk.at[sb0], rows_v.at[0], gsem0)
                pltpu.sync_copy(src_hbm.at[pl.ds(ebase + _CH, _CH)], sb1)
                pltpu.sync_copy(dst_hbm.at[pl.ds(ebase + _CH, _CH)], db1)
                pltpu.async_copy(hblk.at[sb1], rows_v.at[1], gsem1)

                def ebody(t, c):
                    j0 = 2 * t
                    pltpu.make_async_copy(
                        hblk.at[sb0], rows_v.at[0], gsem0).wait()
                    pltpu.async_copy(rows_v.at[0], acc_s.at[db0], ssem0,
                                     add=True)
                    pltpu.make_async_copy(
                        hblk.at[sb1], rows_v.at[1], gsem1).wait()
                    pltpu.async_copy(rows_v.at[1], acc_s.at[db1], ssem1,
                                     add=True)

                    @pl.when(t < _NCH // 2 - 1)
                    def _():
                        e2 = ebase + (j0 + 2) * _CH
                        e3 = ebase + (j0 + 3) * _CH
                        pltpu.sync_copy(src_hbm.at[pl.ds(e2, _CH)], sb0)
                        pltpu.sync_copy(src_hbm.at[pl.ds(e3, _CH)], sb1)
                        pltpu.make_async_copy(rows_v.at[0], acc_s.at[db0],
                                              ssem0).wait()
                        pltpu.sync_copy(dst_hbm.at[pl.ds(e2, _CH)], db0)
                        pltpu.async_copy(hblk.at[sb0], rows_v.at[0], gsem0)
                        pltpu.make_async_copy(rows_v.at[1], acc_s.at[db1],
                                              ssem1).wait()
                        pltpu.sync_copy(dst_hbm.at[pl.ds(e3, _CH)], db1)
                        pltpu.async_copy(hblk.at[sb1], rows_v.at[1], gsem1)

                    return c

                lax.fori_loop(0, _NCH // 2, ebody, 0)
                pltpu.make_async_copy(rows_v.at[0], acc_s.at[db0],
                                      ssem0).wait()
                pltpu.make_async_copy(rows_v.at[1], acc_s.at[db1],
                                      ssem1).wait()
                plsc.subcore_barrier()
                pltpu.sync_copy(acc_s.at[pl.ds(sid * _RPT, _RPT)],
                                out_hbm.at[fb].at[pl.ds(sid * _RPT, _RPT)])

    return pl.kernel(
        body,
        out_type=jax.ShapeDtypeStruct((nb, _NACC, _FB), jnp.float32),
        mesh=_sc_mesh(),
        scratch_types=[
            pltpu.VMEM((_CH,), jnp.int32),
            pltpu.VMEM((_CH,), jnp.int32),
            pltpu.VMEM((_CH,), jnp.int32),
            pltpu.VMEM((_CH,), jnp.int32),
            pltpu.VMEM((2, _CH, _FB), jnp.float32),
            pltpu.VMEM_SHARED((_NACC, _FB), jnp.float32),
            pltpu.SemaphoreType.DMA,
            pltpu.SemaphoreType.DMA,
            pltpu.SemaphoreType.DMA,
            pltpu.SemaphoreType.DMA,
        ],
    )


_prop2 = _make_prop(2)
_prop4 = _make_prop(4)


def _prep_body(x_ref, d0_ref, d1_ref, xb_ref, dinv_ref):
    j = pl.program_id(1)
    dinv = lax.rsqrt(d0_ref[...] + d1_ref[...])
    xb_ref[0] = x_ref[...] * dinv

    @pl.when(j == 0)
    def _():
        dinv_ref[...] = dinv


def _prep(x, d0, d1):
    return pl.pallas_call(
        _prep_body,
        grid=(_N // _BM, 2),
        in_specs=[
            pl.BlockSpec((_BM, _FB), lambda i, j: (i, j)),
            pl.BlockSpec((_BM, 1), lambda i, j: (i, 0)),
            pl.BlockSpec((_BM, 1), lambda i, j: (i, 0)),
        ],
        out_specs=[
            pl.BlockSpec((1, _BM, _FB), lambda i, j: (j, i, 0)),
            pl.BlockSpec((_BM, 1), lambda i, j: (i, 0)),
        ],
        out_shape=[
            jax.ShapeDtypeStruct((2, _N, _FB), jnp.float32),
            jax.ShapeDtypeStruct((_N, 1), jnp.float32),
        ],
    )(x, d0, d1)


def _mm_body(xb_ref, dinv_ref, w_ref, s_ref, c_ref, out_ref, *, nb_in, relu,
             in_dinv, out_dinv, blocked_out):
    x = jnp.concatenate([xb_ref[k] for k in range(nb_in)], axis=1)
    if in_dinv:
        x = x * dinv_ref[...]
    acc = jnp.dot(x, w_ref[...], preferred_element_type=jnp.float32)
    acc = acc * s_ref[...] + c_ref[...]
    if relu:
        acc = jnp.maximum(acc, 0.0)
    if out_dinv:
        acc = acc * dinv_ref[...]
    if blocked_out:
        out_ref[0] = acc
    else:
        out_ref[...] = acc


def _mm_blocked_in(xb, dinv, w, s, c, *, nb_in, nb_out, relu, in_dinv,
                   out_dinv, blocked_out):
    din = nb_in * _FB
    body = functools.partial(_mm_body, nb_in=nb_in, relu=relu,
                             in_dinv=in_dinv, out_dinv=out_dinv,
                             blocked_out=blocked_out)
    if blocked_out:
        out_spec = pl.BlockSpec((1, _BM, _FB), lambda i, j: (j, i, 0))
        out_shape = jax.ShapeDtypeStruct((nb_out, _N, _FB), jnp.float32)
    else:
        out_spec = pl.BlockSpec((_BM, _FB), lambda i, j: (i, j))
        out_shape = jax.ShapeDtypeStruct((_N, nb_out * _FB), jnp.float32)
    return pl.pallas_call(
        body,
        grid=(_N // _BM, nb_out),
        in_specs=[
            pl.BlockSpec((nb_in, _BM, _FB), lambda i, j: (0, i, 0)),
            pl.BlockSpec((_BM, 1), lambda i, j: (i, 0)),
            pl.BlockSpec((din, _FB), lambda i, j: (0, j)),
            pl.BlockSpec((1, _FB), lambda i, j: (0, j)),
            pl.BlockSpec((1, _FB), lambda i, j: (0, j)),
        ],
        out_specs=out_spec,
        out_shape=out_shape,
    )(xb, dinv, w, s, c)


def _mm_plain_body(x_ref, dinv_ref, w_ref, out_ref):
    acc = jnp.dot(x_ref[...], w_ref[...], preferred_element_type=jnp.float32)
    out_ref[0] = acc * dinv_ref[...]


def _mm_plain_in(x, dinv, w, *, din, nb_out):
    return pl.pallas_call(
        _mm_plain_body,
        grid=(_N // _BM, nb_out),
        in_specs=[
            pl.BlockSpec((_BM, din), lambda i, j: (i, 0)),
            pl.BlockSpec((_BM, 1), lambda i, j: (i, 0)),
            pl.BlockSpec((din, _FB), lambda i, j: (0, j)),
        ],
        out_specs=pl.BlockSpec((1, _BM, _FB), lambda i, j: (j, i, 0)),
        out_shape=jax.ShapeDtypeStruct((nb_out, _N, _FB), jnp.float32),
    )(x, dinv, w)


def _fin_body(ab_ref, dinv_ref, b_ref, out_ref):
    out_ref[...] = ab_ref[0] * dinv_ref[...] + b_ref[...]


def _fin(ab, dinv, b):
    return pl.pallas_call(
        _fin_body,
        grid=(_N // _BM, 2),
        in_specs=[
            pl.BlockSpec((1, _BM, _FB), lambda i, j: (j, i, 0)),
            pl.BlockSpec((_BM, 1), lambda i, j: (i, 0)),
            pl.BlockSpec((1, _FB), lambda i, j: (0, j)),
        ],
        out_specs=pl.BlockSpec((_BM, _FB), lambda i, j: (i, j)),
        out_shape=jax.ShapeDtypeStruct((_N, 2 * _FB), jnp.float32),
    )(ab, dinv, b)


def kernel(x, edge_index, W1, b1, g1, beta1, rm1, rv1, W2, b2, g2, beta2,
           rm2, rv2, W3, b3):
    ei = edge_index.astype(jnp.int32)
    loop = jnp.arange(_N, dtype=jnp.int32)
    src = jnp.concatenate([ei[0], loop,
                           jnp.zeros((_EPP - _EP,), jnp.int32)])
    dst = jnp.concatenate([ei[1], loop,
                           jnp.full((_EPP - _EP,), _N, jnp.int32)])

    s1 = (g1 * lax.rsqrt(rv1 + _EPS)).reshape(1, -1)
    c1 = ((b1 - rm1) * s1[0] + beta1).reshape(1, -1)
    s2 = (g2 * lax.rsqrt(rv2 + _EPS)).reshape(1, -1)
    c2 = ((b2 - rm2) * s2[0] + beta2).reshape(1, -1)

    degp = _deg_call(dst)
    d0 = degp[0, :_N].reshape(_N, 1)
    d1 = degp[1, :_N].reshape(_N, 1)

    xb, dinv = _prep(x, d0, d1)
    a0 = _prop2(src, dst, xb)
    h1b = _mm_blocked_in(a0, dinv, W1, s1, c1, nb_in=2, nb_out=4,
                         relu=True, in_dinv=True, out_dinv=True,
                         blocked_out=True)
    a1 = _prop4(src, dst, h1b)
    h2 = _mm_blocked_in(a1, dinv, W2, s2, c2, nb_in=4, nb_out=4,
                        relu=True, in_dinv=True, out_dinv=False,
                        blocked_out=False)
    gb = _mm_plain_in(h2, dinv, W3, din=512, nb_out=2)
    a2 = _prop2(src, dst, gb)
    return _fin(a2, dinv, b3.reshape(1, -1))


# speedup vs baseline: 1.1919x; 1.1919x over previous
import functools

import jax
import jax.numpy as jnp
from jax import lax
from jax.experimental import pallas as pl
from jax.experimental.pallas import tpu as pltpu
from jax.experimental.pallas import tpu_sc as plsc

_N = 10000
_E = 160000
_EPS = 1e-5
_FB = 128
_CH = 128
_NT = 16
_EP = _E + _N
_CHUNK = _NT * _CH
_EPP = ((_EP + _CHUNK - 1) // _CHUNK) * _CHUNK
_PER_TILE = _EPP // _NT
_NCH = _PER_TILE // _CH
_EROWS = _EPP // _CH
_NACC = 10240
_RPT = _NACC // _NT
_BM = 2000


def _sc_mesh():
    return plsc.VectorSubcoreMesh(core_axis_name="c", subcore_axis_name="s")


def _deg_body(dst_hbm, deg_hbm, idx_v, ones_v, zbuf_v, acc_s, sem):
    cid = lax.axis_index("c")
    sid = lax.axis_index("s")
    half = _EPP // 2
    ppt = half // _NT

    def f1(i, c):
        ones_v[pl.ds(i * 16, 16)] = jnp.ones((16,), jnp.float32)
        return c

    lax.fori_loop(0, _CH // 16, f1, 0)

    def f0(i, c):
        zbuf_v[pl.ds(i * 16, 16)] = jnp.zeros((16,), jnp.float32)
        return c

    lax.fori_loop(0, _RPT // 16, f0, 0)
    pltpu.sync_copy(zbuf_v, acc_s.at[pl.ds(sid * _RPT, _RPT)])
    plsc.subcore_barrier()
    ebase = cid * half + sid * ppt

    def body(i, c):
        pltpu.sync_copy(dst_hbm.at[pl.ds(ebase + i * _CH, _CH)], idx_v)
        pltpu.sync_copy(ones_v, acc_s.at[idx_v], add=True)
        return c

    lax.fori_loop(0, ppt // _CH, body, 0)
    plsc.subcore_barrier()
    for c in range(2):
        @pl.when(cid == c)
        def _(c=c):
            pltpu.sync_copy(acc_s.at[pl.ds(sid * _RPT, _RPT)],
                            deg_hbm.at[c].at[pl.ds(sid * _RPT, _RPT)])


_deg_call = pl.kernel(
    _deg_body,
    out_type=jax.ShapeDtypeStruct((2, _NACC), jnp.float32),
    mesh=_sc_mesh(),
    scratch_types=[
        pltpu.VMEM((_CH,), jnp.int32),
        pltpu.VMEM((_CH,), jnp.float32),
        pltpu.VMEM((_RPT,), jnp.float32),
        pltpu.VMEM_SHARED((_NACC,), jnp.float32),
        pltpu.SemaphoreType.DMA,
    ],
)


def _make_prop(nb):
    bpc = nb // 2

    def body(src_hbm, dst_hbm, h_hbm, out_hbm, sb0a, db0a, sb0b, db0b,
             sb1a, db1a, sb1b, db1b, rows_v, acc_s,
             gsem0, gsem1, i0a, i0b, i1a, i1b):
        cid = lax.axis_index("c")
        sid = lax.axis_index("s")
        ebase = sid * _PER_TILE
        nfull = _NCH // 4 - 1

        def iload(e, sb, db, sem):
            pltpu.async_copy(src_hbm.at[pl.ds(e, _CH)], sb, sem)
            pltpu.async_copy(dst_hbm.at[pl.ds(e, _CH)], db, sem)

        def iwait(e, sb, db, sem):
            pltpu.make_async_copy(src_hbm.at[pl.ds(e, _CH)], sb, sem).wait()
            pltpu.make_async_copy(dst_hbm.at[pl.ds(e, _CH)], db, sem).wait()

        for fb in range(nb):
            owner = fb // bpc

            @pl.when(cid == owner)
            def _(fb=fb):
                def zf(i, c):
                    rows_v[0, i // (_FB // 16),
                           pl.ds((i % (_FB // 16)) * 16, 16)] = (
                        jnp.zeros((16,), jnp.float32))
                    return c

                lax.fori_loop(0, _CH * (_FB // 16), zf, 0)
                for z in range(_RPT // _CH):
                    pltpu.sync_copy(
                        rows_v.at[0],
                        acc_s.at[pl.ds(sid * _RPT + z * _CH, _CH)])
                plsc.subcore_barrier()
                hblk = h_hbm.at[fb]
                pltpu.sync_copy(src_hbm.at[pl.ds(ebase, _CH)], sb0a)
                pltpu.sync_copy(dst_hbm.at[pl.ds(ebase, _CH)], db0a)
                pltpu.async_copy(hblk.at[sb0a], rows_v.at[0], gsem0)
                pltpu.sync_copy(src_hbm.at[pl.ds(ebase + _CH, _CH)], sb1a)
                pltpu.sync_copy(dst_hbm.at[pl.ds(ebase + _CH, _CH)], db1a)
                pltpu.async_copy(hblk.at[sb1a], rows_v.at[1], gsem1)
                iload(ebase + 2 * _CH, sb0b, db0b, i0b)
                iload(ebase + 3 * _CH, sb1b, db1b, i1b)

                def step(j, gsem, buf, sb, db, sbn, dbn, isem_n, isem_c):
                    pltpu.make_async_copy(hblk.at[sb], buf, gsem).wait()
                    pltpu.sync_copy(buf, acc_s.at[db], add=True)
                    iload(ebase + (j + 4) * _CH, sb, db, isem_c)
                    iwait(ebase + (j + 2) * _CH, sbn, dbn, isem_n)
                    pltpu.async_copy(hblk.at[sbn], buf, gsem)

                def ebody(t, c):
                    j0 = 4 * t
                    step(j0, gsem0, rows_v.at[0], sb0a, db0a, sb0b, db0b,
                         i0b, i0a)
                    step(j0 + 1, gsem1, rows_v.at[1], sb1a, db1a, sb1b,
                         db1b, i1b, i1a)
                    step(j0 + 2, gsem0, rows_v.at[0], sb0b, db0b, sb0a,
                         db0a, i0a, i0b)
                    step(j0 + 3, gsem1, rows_v.at[1], sb1b, db1b, sb1a,
                         db1a, i1a, i1b)
                    return c

                lax.fori_loop(0, nfull, ebody, 0)
                jl = ebase + (_NCH - 4) * _CH
                pltpu.make_async_copy(hblk.at[sb0a], rows_v.at[0],
                                      gsem0).wait()
                pltpu.sync_copy(rows_v.at[0], acc_s.at[db0a], add=True)
                iwait(jl + 2 * _CH, sb0b, db0b, i0b)
                pltpu.async_copy(hblk.at[sb0b], rows_v.at[0], gsem0)
                pltpu.make_async_copy(hblk.at[sb1a], rows_v.at[1],
                                      gsem1).wait()
                pltpu.sync_copy(rows_v.at[1], acc_s.at[db1a], add=True)
                iwait(jl + 3 * _CH, sb1b, db1b, i1b)
                pltpu.async_copy(hblk.at[sb1b], rows_v.at[1], gsem1)
                pltpu.make_async_copy(hblk.at[sb0b], rows_v.at[0],
                                      gsem0).wait()
                pltpu.sync_copy(rows_v.at[0], acc_s.at[db0b], add=True)
                pltpu.make_async_copy(hblk.at[sb1b], rows_v.at[1],
                                      gsem1).wait()
                pltpu.sync_copy(rows_v.at[1], acc_s.at[db1b], add=True)
                plsc.subcore_barrier()
                pltpu.sync_copy(acc_s.at[pl.ds(sid * _RPT, _RPT)],
                                out_hbm.at[fb].at[pl.ds(sid * _RPT, _RPT)])

    return pl.kernel(
        body,
        out_type=jax.ShapeDtypeStruct((nb, _NACC, _FB), jnp.float32),
        mesh=_sc_mesh(),
        scratch_types=[
            pltpu.VMEM((_CH,), jnp.int32),
            pltpu.VMEM((_CH,), jnp.int32),
            pltpu.VMEM((_CH,), jnp.int32),
            pltpu.VMEM((_CH,), jnp.int32),
            pltpu.VMEM((_CH,), jnp.int32),
            pltpu.VMEM((_CH,), jnp.int32),
            pltpu.VMEM((_CH,), jnp.int32),
            pltpu.VMEM((_CH,), jnp.int32),
            pltpu.VMEM((2, _CH, _FB), jnp.float32),
            pltpu.VMEM_SHARED((_NACC, _FB), jnp.float32),
            pltpu.SemaphoreType.DMA,
            pltpu.SemaphoreType.DMA,
            pltpu.SemaphoreType.DMA,
            pltpu.SemaphoreType.DMA,
            pltpu.SemaphoreType.DMA,
            pltpu.SemaphoreType.DMA,
        ],
    )


_prop2 = _make_prop(2)
_prop4 = _make_prop(4)


def _prep_body(x_ref, d0_ref, d1_ref, xb_ref, dinv_ref):
    j = pl.program_id(1)
    dinv = lax.rsqrt(d0_ref[...] + d1_ref[...])
    xb_ref[0] = x_ref[...] * dinv

    @pl.when(j == 0)
    def _():
        dinv_ref[...] = dinv


def _prep(x, d0, d1):
    return pl.pallas_call(
        _prep_body,
        grid=(_N // _BM, 2),
        in_specs=[
            pl.BlockSpec((_BM, _FB), lambda i, j: (i, j)),
            pl.BlockSpec((_BM, 1), lambda i, j: (i, 0)),
            pl.BlockSpec((_BM, 1), lambda i, j: (i, 0)),
        ],
        out_specs=[
            pl.BlockSpec((1, _BM, _FB), lambda i, j: (j, i, 0)),
            pl.BlockSpec((_BM, 1), lambda i, j: (i, 0)),
        ],
        out_shape=[
            jax.ShapeDtypeStruct((2, _N, _FB), jnp.float32),
            jax.ShapeDtypeStruct((_N, 1), jnp.float32),
        ],
    )(x, d0, d1)


def _mm_body(xb_ref, dinv_ref, w_ref, s_ref, c_ref, out_ref, *, nb_in, relu,
             in_dinv, out_dinv, blocked_out):
    x = jnp.concatenate([xb_ref[k] for k in range(nb_in)], axis=1)
    if in_dinv:
        x = x * dinv_ref[...]
    acc = jnp.dot(x, w_ref[...], preferred_element_type=jnp.float32)
    acc = acc * s_ref[...] + c_ref[...]
    if relu:
        acc = jnp.maximum(acc, 0.0)
    if out_dinv:
        acc = acc * dinv_ref[...]
    if blocked_out:
        out_ref[0] = acc
    else:
        out_ref[...] = acc


def _mm_blocked_in(xb, dinv, w, s, c, *, nb_in, nb_out, relu, in_dinv,
                   out_dinv, blocked_out):
    din = nb_in * _FB
    body = functools.partial(_mm_body, nb_in=nb_in, relu=relu,
                             in_dinv=in_dinv, out_dinv=out_dinv,
                             blocked_out=blocked_out)
    if blocked_out:
        out_spec = pl.BlockSpec((1, _BM, _FB), lambda i, j: (j, i, 0))
        out_shape = jax.ShapeDtypeStruct((nb_out, _N, _FB), jnp.float32)
    else:
        out_spec = pl.BlockSpec((_BM, _FB), lambda i, j: (i, j))
        out_shape = jax.ShapeDtypeStruct((_N, nb_out * _FB), jnp.float32)
    return pl.pallas_call(
        body,
        grid=(_N // _BM, nb_out),
        in_specs=[
            pl.BlockSpec((nb_in, _BM, _FB), lambda i, j: (0, i, 0)),
            pl.BlockSpec((_BM, 1), lambda i, j: (i, 0)),
            pl.BlockSpec((din, _FB), lambda i, j: (0, j)),
            pl.BlockSpec((1, _FB), lambda i, j: (0, j)),
            pl.BlockSpec((1, _FB), lambda i, j: (0, j)),
        ],
        out_specs=out_spec,
        out_shape=out_shape,
    )(xb, dinv, w, s, c)


def _mm_plain_body(x_ref, dinv_ref, w_ref, out_ref):
    acc = jnp.dot(x_ref[...], w_ref[...], preferred_element_type=jnp.float32)
    out_ref[0] = acc * dinv_ref[...]


def _mm_plain_in(x, dinv, w, *, din, nb_out):
    return pl.pallas_call(
        _mm_plain_body,
        grid=(_N // _BM, nb_out),
        in_specs=[
            pl.BlockSpec((_BM, din), lambda i, j: (i, 0)),
            pl.BlockSpec((_BM, 1), lambda i, j: (i, 0)),
            pl.BlockSpec((din, _FB), lambda i, j: (0, j)),
        ],
        out_specs=pl.BlockSpec((1, _BM, _FB), lambda i, j: (j, i, 0)),
        out_shape=jax.ShapeDtypeStruct((nb_out, _N, _FB), jnp.float32),
    )(x, dinv, w)


def _fin_body(ab_ref, dinv_ref, b_ref, out_ref):
    out_ref[...] = ab_ref[0] * dinv_ref[...] + b_ref[...]


def _fin(ab, dinv, b):
    return pl.pallas_call(
        _fin_body,
        grid=(_N // _BM, 2),
        in_specs=[
            pl.BlockSpec((1, _BM, _FB), lambda i, j: (j, i, 0)),
            pl.BlockSpec((_BM, 1), lambda i, j: (i, 0)),
            pl.BlockSpec((1, _FB), lambda i, j: (0, j)),
        ],
        out_specs=pl.BlockSpec((_BM, _FB), lambda i, j: (i, j)),
        out_shape=jax.ShapeDtypeStruct((_N, 2 * _FB), jnp.float32),
    )(ab, dinv, b)


def kernel(x, edge_index, W1, b1, g1, beta1, rm1, rv1, W2, b2, g2, beta2,
           rm2, rv2, W3, b3):
    ei = edge_index.astype(jnp.int32)
    loop = jnp.arange(_N, dtype=jnp.int32)
    src = jnp.concatenate([ei[0], loop,
                           jnp.zeros((_EPP - _EP,), jnp.int32)])
    dst = jnp.concatenate([ei[1], loop,
                           jnp.full((_EPP - _EP,), _N, jnp.int32)])

    s1 = (g1 * lax.rsqrt(rv1 + _EPS)).reshape(1, -1)
    c1 = ((b1 - rm1) * s1[0] + beta1).reshape(1, -1)
    s2 = (g2 * lax.rsqrt(rv2 + _EPS)).reshape(1, -1)
    c2 = ((b2 - rm2) * s2[0] + beta2).reshape(1, -1)

    degp = _deg_call(dst)
    d0 = degp[0, :_N].reshape(_N, 1)
    d1 = degp[1, :_N].reshape(_N, 1)

    xb, dinv = _prep(x, d0, d1)
    a0 = _prop2(src, dst, xb)
    h1b = _mm_blocked_in(a0, dinv, W1, s1, c1, nb_in=2, nb_out=4,
                         relu=True, in_dinv=True, out_dinv=True,
                         blocked_out=True)
    a1 = _prop4(src, dst, h1b)
    h2 = _mm_blocked_in(a1, dinv, W2, s2, c2, nb_in=4, nb_out=4,
                        relu=True, in_dinv=True, out_dinv=False,
                        blocked_out=False)
    gb = _mm_plain_in(h2, dinv, W3, din=512, nb_out=2)
    a2 = _prop2(src, dst, gb)
    return _fin(a2, dinv, b3.reshape(1, -1))


# speedup vs baseline: 1.2304x; 1.0323x over previous
import functools

import jax
import jax.numpy as jnp
from jax import lax
from jax.experimental import pallas as pl
from jax.experimental.pallas import tpu as pltpu
from jax.experimental.pallas import tpu_sc as plsc

_N = 10000
_E = 160000
_EPS = 1e-5
_FB = 128
_CH = 128
_NT = 16
_EP = _E + _N
_CHUNK = _NT * _CH
_EPP = ((_EP + _CHUNK - 1) // _CHUNK) * _CHUNK
_PER_TILE = _EPP // _NT
_NCH = _PER_TILE // _CH
_EROWS = _EPP // _CH
_NACC = 10112
_RPT = _NACC // _NT
_BM = 2000


def _sc_mesh():
    return plsc.VectorSubcoreMesh(core_axis_name="c", subcore_axis_name="s")


_NACC_D = 10240
_RPT_D = _NACC_D // _NT


def _deg_body(dst_hbm, deg_hbm, idx_v, ones_v, zbuf_v, acc_s, sem):
    cid = lax.axis_index("c")
    sid = lax.axis_index("s")
    half = _EPP // 2
    ppt = half // _NT

    def f1(i, c):
        ones_v[pl.ds(i * 16, 16)] = jnp.ones((16,), jnp.float32)
        return c

    lax.fori_loop(0, _CH // 16, f1, 0)

    def f0(i, c):
        zbuf_v[pl.ds(i * 16, 16)] = jnp.zeros((16,), jnp.float32)
        return c

    lax.fori_loop(0, _RPT_D // 16, f0, 0)
    pltpu.sync_copy(zbuf_v, acc_s.at[pl.ds(sid * _RPT_D, _RPT_D)])
    plsc.subcore_barrier()
    ebase = cid * half + sid * ppt

    def body(i, c):
        pltpu.sync_copy(dst_hbm.at[pl.ds(ebase + i * _CH, _CH)], idx_v)
        pltpu.sync_copy(ones_v, acc_s.at[idx_v], add=True)
        return c

    lax.fori_loop(0, ppt // _CH, body, 0)
    plsc.subcore_barrier()
    for c in range(2):
        @pl.when(cid == c)
        def _(c=c):
            pltpu.sync_copy(acc_s.at[pl.ds(sid * _RPT_D, _RPT_D)],
                            deg_hbm.at[c].at[pl.ds(sid * _RPT_D, _RPT_D)])


_deg_call = pl.kernel(
    _deg_body,
    out_type=jax.ShapeDtypeStruct((2, _NACC_D), jnp.float32),
    mesh=_sc_mesh(),
    scratch_types=[
        pltpu.VMEM((_CH,), jnp.int32),
        pltpu.VMEM((_CH,), jnp.float32),
        pltpu.VMEM((_RPT_D,), jnp.float32),
        pltpu.VMEM_SHARED((_NACC_D,), jnp.float32),
        pltpu.SemaphoreType.DMA,
    ],
)


def _make_prop(nb):
    bpc = nb // 2

    def body(src_hbm, dst_hbm, h_hbm, out_hbm, sb0, sb1, sb2, db0, db1, db2,
             rows_v, acc_s, g0, g1, g2, s0, s1, s2, is0, is1, is2,
             id0, id1, id2):
        cid = lax.axis_index("c")
        sid = lax.axis_index("s")
        ebase = sid * _PER_TILE
        sbs = (sb0, sb1, sb2)
        dbs = (db0, db1, db2)
        gs = (g0, g1, g2)
        ss = (s0, s1, s2)
        iss = (is0, is1, is2)
        ids = (id0, id1, id2)

        for fb in range(nb):
            owner = fb // bpc

            @pl.when(cid == owner)
            def _(fb=fb):
                def zf(i, c):
                    rows_v[0, i // (_FB // 16),
                           pl.ds((i % (_FB // 16)) * 16, 16)] = (
                        jnp.zeros((16,), jnp.float32))
                    return c

                lax.fori_loop(0, _CH * (_FB // 16), zf, 0)
                for z in range(_RPT // _CH):
                    pltpu.sync_copy(
                        rows_v.at[0],
                        acc_s.at[pl.ds(sid * _RPT + z * _CH, _CH)])
                rem = _RPT % _CH
                if rem:
                    pltpu.sync_copy(
                        rows_v.at[0].at[pl.ds(0, rem)],
                        acc_s.at[pl.ds(sid * _RPT + (_RPT // _CH) * _CH,
                                       rem)])
                plsc.subcore_barrier()
                hblk = h_hbm.at[fb]

                def step(j, p, w_ssem=True, do_dst=True, do_gather=True,
                         do_src=True):
                    r = p
                    o = (p + 2) % 3
                    pltpu.make_async_copy(hblk.at[sbs[r]], rows_v.at[r],
                                          gs[r]).wait()
                    pltpu.make_async_copy(
                        dst_hbm.at[pl.ds(ebase + j * _CH, _CH)],
                        dbs[r], ids[r]).wait()
                    pltpu.async_copy(rows_v.at[r], acc_s.at[dbs[r]], ss[r],
                                     add=True)
                    if w_ssem:
                        pltpu.make_async_copy(rows_v.at[o], acc_s.at[dbs[o]],
                                              ss[o]).wait()
                    if do_dst:
                        pltpu.async_copy(
                            dst_hbm.at[pl.ds(ebase + (j + 2) * _CH, _CH)],
                            dbs[o], ids[o])
                    if do_gather:
                        pltpu.make_async_copy(
                            src_hbm.at[pl.ds(ebase + (j + 2) * _CH, _CH)],
                            sbs[o], iss[o]).wait()
                        pltpu.async_copy(hblk.at[sbs[o]], rows_v.at[o],
                                         gs[o])
                    if do_src:
                        pltpu.async_copy(
                            src_hbm.at[pl.ds(ebase + (j + 3) * _CH, _CH)],
                            sbs[r], iss[r])

                for k in range(3):
                    e = ebase + k * _CH
                    pltpu.async_copy(src_hbm.at[pl.ds(e, _CH)], sbs[k],
                                     iss[k])
                    pltpu.async_copy(dst_hbm.at[pl.ds(e, _CH)], dbs[k],
                                     ids[k])
                for k in range(3):
                    pltpu.make_async_copy(
                        src_hbm.at[pl.ds(ebase + k * _CH, _CH)],
                        sbs[k], iss[k]).wait()
                    pltpu.async_copy(hblk.at[sbs[k]], rows_v.at[k], gs[k])
                step(0, 0, w_ssem=False, do_dst=False, do_gather=False)

                def ebody(t, c):
                    j1 = 3 * t + 1
                    step(j1, 1)
                    step(j1 + 1, 2)
                    step(j1 + 2, 0)
                    return c

                lax.fori_loop(0, (_NCH - 6) // 3, ebody, 0)
                step(_NCH - 5, (_NCH - 5) % 3)
                step(_NCH - 4, (_NCH - 4) % 3)
                step(_NCH - 3, (_NCH - 3) % 3, do_src=False)
                step(_NCH - 2, (_NCH - 2) % 3, do_dst=False,
                     do_gather=False, do_src=False)
                step(_NCH - 1, (_NCH - 1) % 3, do_dst=False,
                     do_gather=False, do_src=False)
                p_last = (_NCH - 1) % 3
                pltpu.make_async_copy(rows_v.at[p_last],
                                      acc_s.at[dbs[p_last]],
                                      ss[p_last]).wait()
                plsc.subcore_barrier()
                pltpu.sync_copy(acc_s.at[pl.ds(sid * _RPT, _RPT)],
                                out_hbm.at[fb].at[pl.ds(sid * _RPT, _RPT)])

    return pl.kernel(
        body,
        out_type=jax.ShapeDtypeStruct((nb, _NACC, _FB), jnp.float32),
        mesh=_sc_mesh(),
        scratch_types=(
            [pltpu.VMEM((_CH,), jnp.int32)] * 6
            + [pltpu.VMEM((3, _CH, _FB), jnp.float32),
               pltpu.VMEM_SHARED((_NACC, _FB), jnp.float32)]
            + [pltpu.SemaphoreType.DMA] * 12
        ),
    )


_prop2 = _make_prop(2)
_prop4 = _make_prop(4)


def _prep_body(x_ref, d0_ref, d1_ref, xb_ref, dinv_ref):
    j = pl.program_id(1)
    dinv = lax.rsqrt(d0_ref[...] + d1_ref[...])
    xb_ref[0] = x_ref[...] * dinv

    @pl.when(j == 0)
    def _():
        dinv_ref[...] = dinv


def _prep(x, d0, d1):
    return pl.pallas_call(
        _prep_body,
        grid=(_N // _BM, 2),
        in_specs=[
            pl.BlockSpec((_BM, _FB), lambda i, j: (i, j)),
            pl.BlockSpec((_BM, 1), lambda i, j: (i, 0)),
            pl.BlockSpec((_BM, 1), lambda i, j: (i, 0)),
        ],
        out_specs=[
            pl.BlockSpec((1, _BM, _FB), lambda i, j: (j, i, 0)),
            pl.BlockSpec((_BM, 1), lambda i, j: (i, 0)),
        ],
        out_shape=[
            jax.ShapeDtypeStruct((2, _N, _FB), jnp.float32),
            jax.ShapeDtypeStruct((_N, 1), jnp.float32),
        ],
    )(x, d0, d1)


def _mm_body(xb_ref, dinv_ref, w_ref, s_ref, c_ref, out_ref, *, nb_in, relu,
             in_dinv, out_dinv, blocked_out):
    x = jnp.concatenate([xb_ref[k] for k in range(nb_in)], axis=1)
    if in_dinv:
        x = x * dinv_ref[...]
    acc = jnp.dot(x, w_ref[...], preferred_element_type=jnp.float32)
    acc = acc * s_ref[...] + c_ref[...]
    if relu:
        acc = jnp.maximum(acc, 0.0)
    if out_dinv:
        acc = acc * dinv_ref[...]
    if blocked_out:
        out_ref[0] = acc
    else:
        out_ref[...] = acc


def _mm_blocked_in(xb, dinv, w, s, c, *, nb_in, nb_out, relu, in_dinv,
                   out_dinv, blocked_out):
    din = nb_in * _FB
    body = functools.partial(_mm_body, nb_in=nb_in, relu=relu,
                             in_dinv=in_dinv, out_dinv=out_dinv,
                             blocked_out=blocked_out)
    if blocked_out:
        out_spec = pl.BlockSpec((1, _BM, _FB), lambda i, j: (j, i, 0))
        out_shape = jax.ShapeDtypeStruct((nb_out, _N, _FB), jnp.float32)
    else:
        out_spec = pl.BlockSpec((_BM, _FB), lambda i, j: (i, j))
        out_shape = jax.ShapeDtypeStruct((_N, nb_out * _FB), jnp.float32)
    return pl.pallas_call(
        body,
        grid=(_N // _BM, nb_out),
        in_specs=[
            pl.BlockSpec((nb_in, _BM, _FB), lambda i, j: (0, i, 0)),
            pl.BlockSpec((_BM, 1), lambda i, j: (i, 0)),
            pl.BlockSpec((din, _FB), lambda i, j: (0, j)),
            pl.BlockSpec((1, _FB), lambda i, j: (0, j)),
            pl.BlockSpec((1, _FB), lambda i, j: (0, j)),
        ],
        out_specs=out_spec,
        out_shape=out_shape,
    )(xb, dinv, w, s, c)


def _mm_plain_body(x_ref, dinv_ref, w_ref, out_ref):
    acc = jnp.dot(x_ref[...], w_ref[...], preferred_element_type=jnp.float32)
    out_ref[0] = acc * dinv_ref[...]


def _mm_plain_in(x, dinv, w, *, din, nb_out):
    return pl.pallas_call(
        _mm_plain_body,
        grid=(_N // _BM, nb_out),
        in_specs=[
            pl.BlockSpec((_BM, din), lambda i, j: (i, 0)),
            pl.BlockSpec((_BM, 1), lambda i, j: (i, 0)),
            pl.BlockSpec((din, _FB), lambda i, j: (0, j)),
        ],
        out_specs=pl.BlockSpec((1, _BM, _FB), lambda i, j: (j, i, 0)),
        out_shape=jax.ShapeDtypeStruct((nb_out, _N, _FB), jnp.float32),
    )(x, dinv, w)


def _fin_body(ab_ref, dinv_ref, b_ref, out_ref):
    out_ref[...] = ab_ref[0] * dinv_ref[...] + b_ref[...]


def _fin(ab, dinv, b):
    return pl.pallas_call(
        _fin_body,
        grid=(_N // _BM, 2),
        in_specs=[
            pl.BlockSpec((1, _BM, _FB), lambda i, j: (j, i, 0)),
            pl.BlockSpec((_BM, 1), lambda i, j: (i, 0)),
            pl.BlockSpec((1, _FB), lambda i, j: (0, j)),
        ],
        out_specs=pl.BlockSpec((_BM, _FB), lambda i, j: (i, j)),
        out_shape=jax.ShapeDtypeStruct((_N, 2 * _FB), jnp.float32),
    )(ab, dinv, b)


def kernel(x, edge_index, W1, b1, g1, beta1, rm1, rv1, W2, b2, g2, beta2,
           rm2, rv2, W3, b3):
    ei = edge_index.astype(jnp.int32)
    loop = jnp.arange(_N, dtype=jnp.int32)
    src = jnp.concatenate([ei[0], loop,
                           jnp.zeros((_EPP - _EP,), jnp.int32)])
    dst = jnp.concatenate([ei[1], loop,
                           jnp.full((_EPP - _EP,), _N, jnp.int32)])

    s1 = (g1 * lax.rsqrt(rv1 + _EPS)).reshape(1, -1)
    c1 = ((b1 - rm1) * s1[0] + beta1).reshape(1, -1)
    s2 = (g2 * lax.rsqrt(rv2 + _EPS)).reshape(1, -1)
    c2 = ((b2 - rm2) * s2[0] + beta2).reshape(1, -1)

    degp = _deg_call(dst)
    d0 = degp[0, :_N].reshape(_N, 1)
    d1 = degp[1, :_N].reshape(_N, 1)

    xb, dinv = _prep(x, d0, d1)
    a0 = _prop2(src, dst, xb)
    h1b = _mm_blocked_in(a0, dinv, W1, s1, c1, nb_in=2, nb_out=4,
                         relu=True, in_dinv=True, out_dinv=True,
                         blocked_out=True)
    a1 = _prop4(src, dst, h1b)
    h2 = _mm_blocked_in(a1, dinv, W2, s2, c2, nb_in=4, nb_out=4,
                        relu=True, in_dinv=True, out_dinv=False,
                        blocked_out=False)
    gb = _mm_plain_in(h2, dinv, W3, din=512, nb_out=2)
    a2 = _prop2(src, dst, gb)
    return _fin(a2, dinv, b3.reshape(1, -1))


# speedup vs baseline: 1.2705x; 1.0326x over previous
import functools

import jax
import jax.numpy as jnp
from jax import lax
from jax.experimental import pallas as pl
from jax.experimental.pallas import tpu as pltpu
from jax.experimental.pallas import tpu_sc as plsc

_N = 10000
_E = 160000
_EPS = 1e-5
_FB = 128
_CH = 128
_NT = 16
_EP = _E + _N
_CHUNK = _NT * _CH
_EPP = ((_EP + _CHUNK - 1) // _CHUNK) * _CHUNK
_PER_TILE = _EPP // _NT
_NCH = _PER_TILE // _CH
_EROWS = _EPP // _CH
_NACC = 10112
_RPT = _NACC // _NT
_BM = 2000


def _sc_mesh():
    return plsc.VectorSubcoreMesh(core_axis_name="c", subcore_axis_name="s")


_NACC_D = 10240
_RPT_D = _NACC_D // _NT


def _deg_body(dst_hbm, deg_hbm, idx_v, ones_v, zbuf_v, acc_s, sem):
    cid = lax.axis_index("c")
    sid = lax.axis_index("s")
    half = _EPP // 2
    ppt = half // _NT

    def f1(i, c):
        ones_v[pl.ds(i * 16, 16)] = jnp.ones((16,), jnp.float32)
        return c

    lax.fori_loop(0, _CH // 16, f1, 0)

    def f0(i, c):
        zbuf_v[pl.ds(i * 16, 16)] = jnp.zeros((16,), jnp.float32)
        return c

    lax.fori_loop(0, _RPT_D // 16, f0, 0)
    pltpu.sync_copy(zbuf_v, acc_s.at[pl.ds(sid * _RPT_D, _RPT_D)])
    plsc.subcore_barrier()
    ebase = cid * half + sid * ppt

    def body(i, c):
        pltpu.sync_copy(dst_hbm.at[pl.ds(ebase + i * _CH, _CH)], idx_v)
        pltpu.sync_copy(ones_v, acc_s.at[idx_v], add=True)
        return c

    lax.fori_loop(0, ppt // _CH, body, 0)
    plsc.subcore_barrier()
    for c in range(2):
        @pl.when(cid == c)
        def _(c=c):
            pltpu.sync_copy(acc_s.at[pl.ds(sid * _RPT_D, _RPT_D)],
                            deg_hbm.at[c].at[pl.ds(sid * _RPT_D, _RPT_D)])


_deg_call = pl.kernel(
    _deg_body,
    out_type=jax.ShapeDtypeStruct((2, _NACC_D), jnp.float32),
    mesh=_sc_mesh(),
    scratch_types=[
        pltpu.VMEM((_CH,), jnp.int32),
        pltpu.VMEM((_CH,), jnp.float32),
        pltpu.VMEM((_RPT_D,), jnp.float32),
        pltpu.VMEM_SHARED((_NACC_D,), jnp.float32),
        pltpu.SemaphoreType.DMA,
    ],
)


def _make_prop(nb):
    bpc = nb // 2

    def body(src_hbm, dst_hbm, h_hbm, out_hbm, sb0, sb1, sb2, db0, db1, db2,
             rows_v, acc_s, g0, g1, g2, s0, s1, s2, is0, is1, is2,
             id0, id1, id2):
        cid = lax.axis_index("c")
        sid = lax.axis_index("s")
        ebase = sid * _PER_TILE
        sbs = (sb0, sb1, sb2)
        dbs = (db0, db1, db2)
        gs = (g0, g1, g2)
        ss = (s0, s1, s2)
        iss = (is0, is1, is2)
        ids = (id0, id1, id2)

        for fb in range(nb):
            owner = fb // bpc

            @pl.when(cid == owner)
            def _(fb=fb):
                def zf(i, c):
                    rows_v[0, i // (_FB // 16),
                           pl.ds((i % (_FB // 16)) * 16, 16)] = (
                        jnp.zeros((16,), jnp.float32))
                    return c

                lax.fori_loop(0, _CH * (_FB // 16), zf, 0)
                for z in range(_RPT // _CH):
                    pltpu.sync_copy(
                        rows_v.at[0],
                        acc_s.at[pl.ds(sid * _RPT + z * _CH, _CH)])
                rem = _RPT % _CH
                if rem:
                    pltpu.sync_copy(
                        rows_v.at[0].at[pl.ds(0, rem)],
                        acc_s.at[pl.ds(sid * _RPT + (_RPT // _CH) * _CH,
                                       rem)])
                plsc.subcore_barrier()
                hblk = h_hbm.at[fb]

                def step(j, p, w_ssem=True, do_dst=True, do_gather=True,
                         do_src=True):
                    r = p
                    o = (p + 2) % 3
                    pltpu.make_async_copy(hblk.at[sbs[r]], rows_v.at[r],
                                          gs[r]).wait()
                    pltpu.make_async_copy(
                        dst_hbm.at[pl.ds(ebase + j * _CH, _CH)],
                        dbs[r], ids[r]).wait()
                    pltpu.async_copy(rows_v.at[r], acc_s.at[dbs[r]], ss[r],
                                     add=True)
                    if w_ssem:
                        pltpu.make_async_copy(rows_v.at[o], acc_s.at[dbs[o]],
                                              ss[o]).wait()
                    if do_dst:
                        pltpu.async_copy(
                            dst_hbm.at[pl.ds(ebase + (j + 2) * _CH, _CH)],
                            dbs[o], ids[o])
                    if do_gather:
                        pltpu.make_async_copy(
                            src_hbm.at[pl.ds(ebase + (j + 2) * _CH, _CH)],
                            sbs[o], iss[o]).wait()
                        pltpu.async_copy(hblk.at[sbs[o]], rows_v.at[o],
                                         gs[o])
                    if do_src:
                        pltpu.async_copy(
                            src_hbm.at[pl.ds(ebase + (j + 3) * _CH, _CH)],
                            sbs[r], iss[r])

                for k in range(3):
                    e = ebase + k * _CH
                    pltpu.async_copy(src_hbm.at[pl.ds(e, _CH)], sbs[k],
                                     iss[k])
                    pltpu.async_copy(dst_hbm.at[pl.ds(e, _CH)], dbs[k],
                                     ids[k])
                for k in range(3):
                    pltpu.make_async_copy(
                        src_hbm.at[pl.ds(ebase + k * _CH, _CH)],
                        sbs[k], iss[k]).wait()
                    pltpu.async_copy(hblk.at[sbs[k]], rows_v.at[k], gs[k])
                step(0, 0, w_ssem=False, do_dst=False, do_gather=False)

                def ebody(t, c):
                    j1 = 3 * t + 1
                    step(j1, 1)
                    step(j1 + 1, 2)
                    step(j1 + 2, 0)
                    return c

                lax.fori_loop(0, (_NCH - 6) // 3, ebody, 0)
                step(_NCH - 5, (_NCH - 5) % 3)
                step(_NCH - 4, (_NCH - 4) % 3)
                step(_NCH - 3, (_NCH - 3) % 3, do_src=False)
                step(_NCH - 2, (_NCH - 2) % 3, do_dst=False,
                     do_gather=False, do_src=False)
                step(_NCH - 1, (_NCH - 1) % 3, do_dst=False,
                     do_gather=False, do_src=False)
                p_last = (_NCH - 1) % 3
                pltpu.make_async_copy(rows_v.at[p_last],
                                      acc_s.at[dbs[p_last]],
                                      ss[p_last]).wait()
                plsc.subcore_barrier()
                pltpu.sync_copy(acc_s.at[pl.ds(sid * _RPT, _RPT)],
                                out_hbm.at[fb].at[pl.ds(sid * _RPT, _RPT)])

    return pl.kernel(
        body,
        out_type=jax.ShapeDtypeStruct((nb, _NACC, _FB), jnp.float32),
        mesh=_sc_mesh(),
        scratch_types=(
            [pltpu.VMEM((_CH,), jnp.int32)] * 6
            + [pltpu.VMEM((3, _CH, _FB), jnp.float32),
               pltpu.VMEM_SHARED((_NACC, _FB), jnp.float32)]
            + [pltpu.SemaphoreType.DMA] * 12
        ),
    )


_prop2 = _make_prop(2)
_prop4 = _make_prop(4)


def _prep_body(x_ref, d0_ref, d1_ref, xb_ref, dinv_ref):
    j = pl.program_id(1)
    dinv = lax.rsqrt(d0_ref[...] + d1_ref[...])
    xb_ref[0] = x_ref[...] * dinv

    @pl.when(j == 0)
    def _():
        dinv_ref[...] = dinv


def _prep(x, d0, d1):
    return pl.pallas_call(
        _prep_body,
        grid=(_N // _BM, 2),
        in_specs=[
            pl.BlockSpec((_BM, _FB), lambda i, j: (i, j)),
            pl.BlockSpec((_BM, 1), lambda i, j: (i, 0)),
            pl.BlockSpec((_BM, 1), lambda i, j: (i, 0)),
        ],
        out_specs=[
            pl.BlockSpec((1, _BM, _FB), lambda i, j: (j, i, 0)),
            pl.BlockSpec((_BM, 1), lambda i, j: (i, 0)),
        ],
        out_shape=[
            jax.ShapeDtypeStruct((2, _N, _FB), jnp.float32),
            jax.ShapeDtypeStruct((_N, 1), jnp.float32),
        ],
    )(x, d0, d1)


def _mm_body(xb_ref, dinv_ref, w_ref, s_ref, c_ref, out_ref, *, nb_in, relu,
             in_dinv, out_dinv, blocked_out):
    x = jnp.concatenate([xb_ref[k] for k in range(nb_in)], axis=1)
    if in_dinv:
        x = x * dinv_ref[...]
    acc = jnp.dot(x, w_ref[...], preferred_element_type=jnp.float32)
    acc = acc * s_ref[...] + c_ref[...]
    if relu:
        acc = jnp.maximum(acc, 0.0)
    if out_dinv:
        acc = acc * dinv_ref[...]
    if blocked_out:
        out_ref[0] = acc
    else:
        out_ref[...] = acc


def _mm_blocked_in(xb, dinv, w, s, c, *, nb_in, nb_out, relu, in_dinv,
                   out_dinv, blocked_out):
    din = nb_in * _FB
    body = functools.partial(_mm_body, nb_in=nb_in, relu=relu,
                             in_dinv=in_dinv, out_dinv=out_dinv,
                             blocked_out=blocked_out)
    if blocked_out:
        out_spec = pl.BlockSpec((1, _BM, _FB), lambda i, j: (j, i, 0))
        out_shape = jax.ShapeDtypeStruct((nb_out, _N, _FB), jnp.float32)
    else:
        out_spec = pl.BlockSpec((_BM, _FB), lambda i, j: (i, j))
        out_shape = jax.ShapeDtypeStruct((_N, nb_out * _FB), jnp.float32)
    return pl.pallas_call(
        body,
        grid=(_N // _BM, nb_out),
        in_specs=[
            pl.BlockSpec((nb_in, _BM, _FB), lambda i, j: (0, i, 0)),
            pl.BlockSpec((_BM, 1), lambda i, j: (i, 0)),
            pl.BlockSpec((din, _FB), lambda i, j: (0, j)),
            pl.BlockSpec((1, _FB), lambda i, j: (0, j)),
            pl.BlockSpec((1, _FB), lambda i, j: (0, j)),
        ],
        out_specs=out_spec,
        out_shape=out_shape,
    )(xb, dinv, w, s, c)


def _mm_plain_body(x_ref, dinv_ref, w_ref, out_ref):
    acc = jnp.dot(x_ref[...], w_ref[...], preferred_element_type=jnp.float32)
    out_ref[0] = acc * dinv_ref[...]


def _mm_plain_in(x, dinv, w, *, din, nb_out):
    return pl.pallas_call(
        _mm_plain_body,
        grid=(_N // _BM, nb_out),
        in_specs=[
            pl.BlockSpec((_BM, din), lambda i, j: (i, 0)),
            pl.BlockSpec((_BM, 1), lambda i, j: (i, 0)),
            pl.BlockSpec((din, _FB), lambda i, j: (0, j)),
        ],
        out_specs=pl.BlockSpec((1, _BM, _FB), lambda i, j: (j, i, 0)),
        out_shape=jax.ShapeDtypeStruct((nb_out, _N, _FB), jnp.float32),
    )(x, dinv, w)


def _mm_fused_body(ab_ref, dinv_ref, w2_ref, s_ref, c_ref, w3_ref, out_ref):
    x = jnp.concatenate([ab_ref[k] for k in range(4)], axis=1)
    x = x * dinv_ref[...]
    t = jnp.dot(x, w2_ref[...], preferred_element_type=jnp.float32)
    t = jnp.maximum(t * s_ref[...] + c_ref[...], 0.0)
    g = jnp.dot(t, w3_ref[...], preferred_element_type=jnp.float32)
    g = g * dinv_ref[...]
    out_ref[0] = g[:, :_FB]
    out_ref[1] = g[:, _FB:]


def _mm_fused(ab, dinv, w2, s, c, w3):
    return pl.pallas_call(
        _mm_fused_body,
        grid=(_N // _BM,),
        in_specs=[
            pl.BlockSpec((4, _BM, _FB), lambda i: (0, i, 0)),
            pl.BlockSpec((_BM, 1), lambda i: (i, 0)),
            pl.BlockSpec((512, 512), lambda i: (0, 0)),
            pl.BlockSpec((1, 512), lambda i: (0, 0)),
            pl.BlockSpec((1, 512), lambda i: (0, 0)),
            pl.BlockSpec((512, 256), lambda i: (0, 0)),
        ],
        out_specs=pl.BlockSpec((2, _BM, _FB), lambda i: (0, i, 0)),
        out_shape=jax.ShapeDtypeStruct((2, _N, _FB), jnp.float32),
    )(ab, dinv, w2, s, c, w3)


def _fin_body(ab_ref, dinv_ref, b_ref, out_ref):
    out_ref[...] = ab_ref[0] * dinv_ref[...] + b_ref[...]


def _fin(ab, dinv, b):
    return pl.pallas_call(
        _fin_body,
        grid=(_N // _BM, 2),
        in_specs=[
            pl.BlockSpec((1, _BM, _FB), lambda i, j: (j, i, 0)),
            pl.BlockSpec((_BM, 1), lambda i, j: (i, 0)),
            pl.BlockSpec((1, _FB), lambda i, j: (0, j)),
        ],
        out_specs=pl.BlockSpec((_BM, _FB), lambda i, j: (i, j)),
        out_shape=jax.ShapeDtypeStruct((_N, 2 * _FB), jnp.float32),
    )(ab, dinv, b)


def kernel(x, edge_index, W1, b1, g1, beta1, rm1, rv1, W2, b2, g2, beta2,
           rm2, rv2, W3, b3):
    ei = edge_index.astype(jnp.int32)
    loop = jnp.arange(_N, dtype=jnp.int32)
    src = jnp.concatenate([ei[0], loop,
                           jnp.zeros((_EPP - _EP,), jnp.int32)])
    dst = jnp.concatenate([ei[1], loop,
                           jnp.full((_EPP - _EP,), _N, jnp.int32)])

    s1 = (g1 * lax.rsqrt(rv1 + _EPS)).reshape(1, -1)
    c1 = ((b1 - rm1) * s1[0] + beta1).reshape(1, -1)
    s2 = (g2 * lax.rsqrt(rv2 + _EPS)).reshape(1, -1)
    c2 = ((b2 - rm2) * s2[0] + beta2).reshape(1, -1)

    degp = _deg_call(dst)
    d0 = degp[0, :_N].reshape(_N, 1)
    d1 = degp[1, :_N].reshape(_N, 1)

    xb, dinv = _prep(x, d0, d1)
    a0 = _prop2(src, dst, xb)
    h1b = _mm_blocked_in(a0, dinv, W1, s1, c1, nb_in=2, nb_out=4,
                         relu=True, in_dinv=True, out_dinv=True,
                         blocked_out=True)
    a1 = _prop4(src, dst, h1b)
    gb = _mm_fused(a1, dinv, W2, s2, c2, W3)
    a2 = _prop2(src, dst, gb)
    return _fin(a2, dinv, b3.reshape(1, -1))
